# baseline (speedup 1.0000x reference)
import jax
import jax.numpy as jnp
from jax import lax
from jax.experimental import pallas as pl
from jax.experimental.pallas import tpu as pltpu
from jax.experimental.pallas import tpu_sc as plsc

N = 10000
E = 320000
C = 128
B = 1024
OUT = 1
TDIM = 16
CP = C + 16
NC, NS = 2, 16
EPS = E // NS
K = 40
NB = 5
RB = 1000
HN, HDN = 5000, 5120
HB, HDB = 512, 544


def _seg_kernel(jobs):
    n = len(jobs)
    mesh = plsc.VectorSubcoreMesh(core_axis_name="c", subcore_axis_name="s")
    out_type = tuple(
        jax.ShapeDtypeStruct((NC, hd, CP), jnp.float32) for _, hd in jobs
    )
    scratch = [
        pltpu.VMEM((EPS,), jnp.int32),
        pltpu.VMEM((EPS,), jnp.int32),
    ] + [pltpu.VMEM((K, CP), jnp.float32) for _ in range(NB)] \
      + [pltpu.SemaphoreType.DMA for _ in range(NB)] \
      + [pltpu.VMEM_SHARED((hd, CP), jnp.float32) for _, hd in jobs]

    def body(*refs):
        tables = refs[:n]
        srcs = refs[n:2 * n]
        dlocs = refs[2 * n:3 * n]
        outs = refs[3 * n:4 * n]
        srcv, dstv = refs[4 * n:4 * n + 2]
        rows = refs[4 * n + 2:4 * n + 2 + NB]
        sems = refs[4 * n + 2 + NB:4 * n + 2 + 2 * NB]
        accs = refs[4 * n + 2 + 2 * NB:]
        rows0 = rows[0]

        c = lax.axis_index("c")
        s = lax.axis_index("s")

        zz = jnp.zeros((16,), jnp.float32)

        def zfill(r, carry):
            for t in range(CP // 16):
                rows0[r, pl.ds(t * 16, 16)] = zz
            return carry
        lax.fori_loop(0, K, zfill, 0)
        for j, (_, hd) in enumerate(jobs):
            spt = hd // NS
            for z0 in range(0, spt, K):
                ln = min(K, spt - z0)
                pltpu.sync_copy(rows0.at[pl.ds(0, ln)],
                                accs[j].at[pl.ds(s * spt + z0, ln)])
        plsc.subcore_barrier()

        for j, (h, hd) in enumerate(jobs):
            pltpu.sync_copy(srcs[j].at[s], srcv)
            pltpu.sync_copy(dlocs[j].at[c].at[s], dstv)

            def gs(g, carry, j=j):
                def sl(b):
                    return pl.ds(g * NB * K + b * K, K)
                cp = pltpu.async_copy(tables[j].at[srcv.at[sl(0)]],
                                      rows[0], sems[0])
                for b in range(NB):
                    cp.wait()
                    if b + 1 < NB:
                        cp = pltpu.async_copy(
                            tables[j].at[srcv.at[sl(b + 1)]],
                            rows[b + 1], sems[b + 1])
                    pltpu.sync_copy(rows[b], accs[j].at[dstv.at[sl(b)]],
                                    add=True)
                return carry
            lax.fori_loop(0, EPS // (NB * K), gs, 0)
        plsc.subcore_barrier()

        for j, (_, hd) in enumerate(jobs):
            spt = hd // NS
            sl = pl.ds(s * spt, spt)
            pltpu.sync_copy(accs[j].at[sl], outs[j].at[c].at[sl])

    return pl.kernel(
        body, out_type=out_type, mesh=mesh, scratch_types=scratch,
        compiler_params=pltpu.CompilerParams(use_tc_tiling_on_sc=False))


_sc1 = _seg_kernel(((HN, HDN), (HB, HDB)))
_sc2 = _seg_kernel(((HB, HDB),))


def _tc1_body(xu, xi, tu, ti, bu, bi, st, fr,
              Weu, beu, Wei, bei, Wt, bt, hu_o, hi_o):
    for x, t, b, We, be, out in ((xu, tu, bu, Weu, beu, hu_o),
                                 (xi, ti, bi, Wei, bei, hi_o)):
        eq = b[...] == lax.broadcasted_iota(jnp.int32, (RB, B), 1)
        rel = (jnp.sum(jnp.where(eq, st[...], 0.0), axis=1, keepdims=True)
               - t[...].astype(jnp.float32))
        pe = jnp.sin(rel * fr[...])
        out[...] = (jnp.maximum(x[...] @ We[...] + be[...], 0.0)
                    + pe @ Wt[...] + bt[...])


_tc1 = pl.pallas_call(
    _tc1_body,
    grid=(N // RB,),
    in_specs=[
        pl.BlockSpec((RB, C), lambda i: (i, 0)),
        pl.BlockSpec((RB, C), lambda i: (i, 0)),
        pl.BlockSpec((RB, 1), lambda i: (i, 0)),
        pl.BlockSpec((RB, 1), lambda i: (i, 0)),
        pl.BlockSpec((RB, 1), lambda i: (i, 0)),
        pl.BlockSpec((RB, 1), lambda i: (i, 0)),
        pl.BlockSpec((1, B), lambda i: (0, 0)),
        pl.BlockSpec((1, TDIM), lambda i: (0, 0)),
        pl.BlockSpec((C, CP), lambda i: (0, 0)),
        pl.BlockSpec((1, CP), lambda i: (0, 0)),
        pl.BlockSpec((C, CP), lambda i: (0, 0)),
        pl.BlockSpec((1, CP), lambda i: (0, 0)),
        pl.BlockSpec((TDIM, CP), lambda i: (0, 0)),
        pl.BlockSpec((1, CP), lambda i: (0, 0)),
    ],
    out_specs=[pl.BlockSpec((RB, CP), lambda i: (i, 0))] * 2,
    out_shape=[jax.ShapeDtypeStruct((N, CP), jnp.float32)] * 2,
)


def _tc2_body(hi, acci, Wr, Wn, colb, out):
    ssum = acci[...][0]
    recip = 1.0 / jnp.maximum(ssum[:, C:C + 1], 1.0)
    mean = ssum * recip
    out[...] = jnp.maximum(hi[...] @ Wr[...] + mean @ Wn[...] + colb[...], 0.0)


_tc2 = pl.pallas_call(
    _tc2_body,
    grid=(N // RB,),
    in_specs=[
        pl.BlockSpec((RB, CP), lambda i: (i, 0)),
        pl.BlockSpec((1, RB, CP), lambda i: (i // 5, i % 5, 0)),
        pl.BlockSpec((CP, CP), lambda i: (0, 0)),
        pl.BlockSpec((CP, CP), lambda i: (0, 0)),
        pl.BlockSpec((1, CP), lambda i: (0, 0)),
    ],
    out_specs=pl.BlockSpec((RB, CP), lambda i: (i, 0)),
    out_shape=jax.ShapeDtypeStruct((N, CP), jnp.float32),
)


def _tc3_body(hu, accu, accu2, Wr1, Wn1, Wr2, Wn2, gam, bet, Wh, bh, out):
    au = accu[...]
    a1 = jnp.concatenate([au[0, :HB], au[1, :HB]], axis=0)
    mu1 = a1 * (1.0 / jnp.maximum(a1[:, C:C + 1], 1.0))
    nh_u = jnp.maximum(hu[...] @ Wr1[...] + mu1 @ Wn1[...], 0.0)
    av = accu2[...]
    a2 = jnp.concatenate([av[0, :HB], av[1, :HB]], axis=0)
    mu2 = a2 * (1.0 / jnp.maximum(a2[:, C:C + 1], 1.0))
    ou = nh_u @ Wr2[...] + mu2 @ Wn2[...]
    m = jnp.mean(ou, axis=0, keepdims=True)
    v = jnp.mean((ou - m) ** 2, axis=0, keepdims=True)
    xn = (ou - m) / jnp.sqrt(v + 1e-5)
    out[...] = (xn * gam[...] + bet[...]) @ Wh[...] + bh[...]


_tc3 = pl.pallas_call(
    _tc3_body,
    grid=(1,),
    in_specs=[
        pl.BlockSpec((B, CP), lambda i: (0, 0)),
        pl.BlockSpec((NC, HDB, CP), lambda i: (0, 0, 0)),
        pl.BlockSpec((NC, HDB, CP), lambda i: (0, 0, 0)),
        pl.BlockSpec((CP, C), lambda i: (0, 0)),
        pl.BlockSpec((CP, C), lambda i: (0, 0)),
        pl.BlockSpec((C, C), lambda i: (0, 0)),
        pl.BlockSpec((CP, C), lambda i: (0, 0)),
        pl.BlockSpec((1, C), lambda i: (0, 0)),
        pl.BlockSpec((1, C), lambda i: (0, 0)),
        pl.BlockSpec((C, OUT), lambda i: (0, 0)),
        pl.BlockSpec((1, OUT), lambda i: (0, 0)),
    ],
    out_specs=pl.BlockSpec((B, OUT), lambda i: (0, 0)),
    out_shape=jax.ShapeDtypeStruct((B, OUT), jnp.float32),
)


def _pad_out(w):
    return jnp.zeros((w.shape[0], CP), jnp.float32).at[:, :C].set(w)


def _pad_rows(w):
    return jnp.zeros((CP, w.shape[1]), jnp.float32).at[:C, :].set(w)


def kernel(x_user, x_item, edge_index_u2i, edge_index_i2u, time_user,
           time_item, seed_time, batch_user, batch_item, W_enc_user,
           b_enc_user, W_enc_item, b_enc_item, W_time, b_time, W_root_u1,
           W_nbr_u1, W_root_i1, W_nbr_i1, W_root_u2, W_nbr_u2, W_root_i2,
           W_nbr_i2, gamma, beta, W_head, b_head):
    del W_root_i2, W_nbr_i2
    f32 = jnp.float32
    freqs = (1.0 / (10000.0 ** (jnp.arange(TDIM, dtype=f32) / TDIM)))
    ones_col = jnp.zeros((1, CP), f32).at[0, C].set(1.0)

    hu, hi = _tc1(
        x_user, x_item,
        time_user.reshape(N, 1), time_item.reshape(N, 1),
        batch_user.reshape(N, 1), batch_item.reshape(N, 1),
        seed_time.astype(f32).reshape(1, B), freqs.reshape(1, TDIM),
        _pad_out(W_enc_user), _pad_out(b_enc_user.reshape(1, C)) + ones_col,
        _pad_out(W_enc_item), _pad_out(b_enc_item.reshape(1, C)) + ones_col,
        _pad_out(W_time), _pad_out(b_time.reshape(1, C)),
    )

    def _localize(dst, h):
        dl = dst[None, :] - (jnp.arange(NC, dtype=jnp.int32) * h)[:, None]
        dl = dl.reshape(NC, NS, EPS)
        dummy = h + jnp.arange(NS, dtype=jnp.int32)[None, :, None]
        return jnp.where((dl >= 0) & (dl < h), dl, dummy)

    su2i = edge_index_u2i[0].reshape(NS, EPS)
    si2u = edge_index_i2u[0].reshape(NS, EPS)
    du2i = _localize(edge_index_u2i[1], HN)
    di2u = _localize(edge_index_i2u[1], HB)

    acc_i, acc_u = _sc1(hu, hi, su2i, si2u, du2i, di2u)

    nhi = _tc2(hi, acc_i, _pad_rows(_pad_out(W_root_i1)),
               _pad_rows(_pad_out(W_nbr_i1)), ones_col)

    acc_u2 = _sc2(nhi, si2u, di2u)
    if isinstance(acc_u2, (tuple, list)):
        (acc_u2,) = acc_u2

    return _tc3(hu, acc_u, acc_u2,
                _pad_rows(W_root_u1), _pad_rows(W_nbr_u1),
                W_root_u2, _pad_rows(W_nbr_u2),
                gamma.reshape(1, C), beta.reshape(1, C),
                W_head, b_head.reshape(1, OUT))


# speedup vs baseline: 1.6240x; 1.6240x over previous
import jax
import jax.numpy as jnp
from jax import lax
from jax.experimental import pallas as pl
from jax.experimental.pallas import tpu as pltpu
from jax.experimental.pallas import tpu_sc as plsc

N = 10000
E = 320000
C = 128
B = 1024
OUT = 1
TDIM = 16
CP = C + 16
NC, NS = 2, 16
EPS = E // (NC * NS)
K = 40
NB = 2
RB = 1000
HDN = N
HDB = B + 16


def _seg_kernel(jobs):
    n = len(jobs)
    mesh = plsc.VectorSubcoreMesh(core_axis_name="c", subcore_axis_name="s")
    out_type = tuple(
        jax.ShapeDtypeStruct((NC, hd, CP), jnp.float32) for _, hd in jobs
    )
    scratch = [
        pltpu.VMEM((EPS,), jnp.int32),
        pltpu.VMEM((EPS,), jnp.int32),
    ] + [pltpu.VMEM((K, CP), jnp.float32) for _ in range(NB)] \
      + [pltpu.SemaphoreType.DMA for _ in range(NB)] \
      + [pltpu.VMEM_SHARED((hd, CP), jnp.float32) for _, hd in jobs]

    def body(*refs):
        tables = refs[:n]
        srcs = refs[n:2 * n]
        dlocs = refs[2 * n:3 * n]
        outs = refs[3 * n:4 * n]
        srcv, dstv = refs[4 * n:4 * n + 2]
        rows = refs[4 * n + 2:4 * n + 2 + NB]
        sems = refs[4 * n + 2 + NB:4 * n + 2 + 2 * NB]
        accs = refs[4 * n + 2 + 2 * NB:]
        rows0 = rows[0]

        c = lax.axis_index("c")
        s = lax.axis_index("s")

        zz = jnp.zeros((16,), jnp.float32)

        def zfill(r, carry):
            for t in range(CP // 16):
                rows0[r, pl.ds(t * 16, 16)] = zz
            return carry
        lax.fori_loop(0, K, zfill, 0)
        for j, (_, hd) in enumerate(jobs):
            spt = hd // NS
            for z0 in range(0, spt, K):
                ln = min(K, spt - z0)
                pltpu.sync_copy(rows0.at[pl.ds(0, ln)],
                                accs[j].at[pl.ds(s * spt + z0, ln)])
        plsc.subcore_barrier()

        for j, (h, hd) in enumerate(jobs):
            pltpu.sync_copy(srcs[j].at[c].at[s], srcv)
            pltpu.sync_copy(dlocs[j].at[c].at[s], dstv)

            def gs(g, carry, j=j):
                def sl(b):
                    return pl.ds(g * NB * K + b * K, K)
                cp = pltpu.async_copy(tables[j].at[srcv.at[sl(0)]],
                                      rows[0], sems[0])
                for b in range(NB):
                    cp.wait()
                    if b + 1 < NB:
                        cp = pltpu.async_copy(
                            tables[j].at[srcv.at[sl(b + 1)]],
                            rows[b + 1], sems[b + 1])
                    pltpu.sync_copy(rows[b], accs[j].at[dstv.at[sl(b)]],
                                    add=True)
                return carry
            lax.fori_loop(0, EPS // (NB * K), gs, 0)
        plsc.subcore_barrier()

        for j, (_, hd) in enumerate(jobs):
            spt = hd // NS
            sl = pl.ds(s * spt, spt)
            pltpu.sync_copy(accs[j].at[sl], outs[j].at[c].at[sl])

    return pl.kernel(
        body, out_type=out_type, mesh=mesh, scratch_types=scratch,
        compiler_params=pltpu.CompilerParams(use_tc_tiling_on_sc=False))


_sc1 = _seg_kernel(((N, HDN), (B, HDB)))
_sc2 = _seg_kernel(((B, HDB),))


def _tc1_body(xu, xi, tu, ti, bu, bi, st, fr,
              Weu, beu, Wei, bei, Wt, bt, hu_o, hi_o):
    for x, t, b, We, be, out in ((xu, tu, bu, Weu, beu, hu_o),
                                 (xi, ti, bi, Wei, bei, hi_o)):
        eq = b[...] == lax.broadcasted_iota(jnp.int32, (RB, B), 1)
        rel = (jnp.sum(jnp.where(eq, st[...], 0.0), axis=1, keepdims=True)
               - t[...].astype(jnp.float32))
        pe = jnp.sin(rel * fr[...])
        out[...] = (jnp.maximum(x[...] @ We[...] + be[...], 0.0)
                    + pe @ Wt[...] + bt[...])


_tc1 = pl.pallas_call(
    _tc1_body,
    grid=(N // RB,),
    in_specs=[
        pl.BlockSpec((RB, C), lambda i: (i, 0)),
        pl.BlockSpec((RB, C), lambda i: (i, 0)),
        pl.BlockSpec((RB, 1), lambda i: (i, 0)),
        pl.BlockSpec((RB, 1), lambda i: (i, 0)),
        pl.BlockSpec((RB, 1), lambda i: (i, 0)),
        pl.BlockSpec((RB, 1), lambda i: (i, 0)),
        pl.BlockSpec((1, B), lambda i: (0, 0)),
        pl.BlockSpec((1, TDIM), lambda i: (0, 0)),
        pl.BlockSpec((C, CP), lambda i: (0, 0)),
        pl.BlockSpec((1, CP), lambda i: (0, 0)),
        pl.BlockSpec((C, CP), lambda i: (0, 0)),
        pl.BlockSpec((1, CP), lambda i: (0, 0)),
        pl.BlockSpec((TDIM, CP), lambda i: (0, 0)),
        pl.BlockSpec((1, CP), lambda i: (0, 0)),
    ],
    out_specs=[pl.BlockSpec((RB, CP), lambda i: (i, 0))] * 2,
    out_shape=[jax.ShapeDtypeStruct((N, CP), jnp.float32)] * 2,
)


def _tc2_body(hi, acci, Wr, Wn, colb, out):
    a = acci[...]
    ssum = a[0] + a[1]
    recip = 1.0 / jnp.maximum(ssum[:, C:C + 1], 1.0)
    mean = ssum * recip
    out[...] = jnp.maximum(hi[...] @ Wr[...] + mean @ Wn[...] + colb[...], 0.0)


_tc2 = pl.pallas_call(
    _tc2_body,
    grid=(N // RB,),
    in_specs=[
        pl.BlockSpec((RB, CP), lambda i: (i, 0)),
        pl.BlockSpec((NC, RB, CP), lambda i: (0, i, 0)),
        pl.BlockSpec((CP, CP), lambda i: (0, 0)),
        pl.BlockSpec((CP, CP), lambda i: (0, 0)),
        pl.BlockSpec((1, CP), lambda i: (0, 0)),
    ],
    out_specs=pl.BlockSpec((RB, CP), lambda i: (i, 0)),
    out_shape=jax.ShapeDtypeStruct((N, CP), jnp.float32),
)


def _tc3_body(hu, accu, accu2, Wr1, Wn1, Wr2, Wn2, gam, bet, Wh, bh, out):
    au = accu[...]
    a1 = au[0, :B] + au[1, :B]
    mu1 = a1 * (1.0 / jnp.maximum(a1[:, C:C + 1], 1.0))
    nh_u = jnp.maximum(hu[...] @ Wr1[...] + mu1 @ Wn1[...], 0.0)
    av = accu2[...]
    a2 = av[0, :B] + av[1, :B]
    mu2 = a2 * (1.0 / jnp.maximum(a2[:, C:C + 1], 1.0))
    ou = nh_u @ Wr2[...] + mu2 @ Wn2[...]
    m = jnp.mean(ou, axis=0, keepdims=True)
    v = jnp.mean((ou - m) ** 2, axis=0, keepdims=True)
    xn = (ou - m) / jnp.sqrt(v + 1e-5)
    out[...] = (xn * gam[...] + bet[...]) @ Wh[...] + bh[...]


_tc3 = pl.pallas_call(
    _tc3_body,
    grid=(1,),
    in_specs=[
        pl.BlockSpec((B, CP), lambda i: (0, 0)),
        pl.BlockSpec((NC, HDB, CP), lambda i: (0, 0, 0)),
        pl.BlockSpec((NC, HDB, CP), lambda i: (0, 0, 0)),
        pl.BlockSpec((CP, C), lambda i: (0, 0)),
        pl.BlockSpec((CP, C), lambda i: (0, 0)),
        pl.BlockSpec((C, C), lambda i: (0, 0)),
        pl.BlockSpec((CP, C), lambda i: (0, 0)),
        pl.BlockSpec((1, C), lambda i: (0, 0)),
        pl.BlockSpec((1, C), lambda i: (0, 0)),
        pl.BlockSpec((C, OUT), lambda i: (0, 0)),
        pl.BlockSpec((1, OUT), lambda i: (0, 0)),
    ],
    out_specs=pl.BlockSpec((B, OUT), lambda i: (0, 0)),
    out_shape=jax.ShapeDtypeStruct((B, OUT), jnp.float32),
)


def _pad_out(w):
    return jnp.zeros((w.shape[0], CP), jnp.float32).at[:, :C].set(w)


def _pad_rows(w):
    return jnp.zeros((CP, w.shape[1]), jnp.float32).at[:C, :].set(w)


def kernel(x_user, x_item, edge_index_u2i, edge_index_i2u, time_user,
           time_item, seed_time, batch_user, batch_item, W_enc_user,
           b_enc_user, W_enc_item, b_enc_item, W_time, b_time, W_root_u1,
           W_nbr_u1, W_root_i1, W_nbr_i1, W_root_u2, W_nbr_u2, W_root_i2,
           W_nbr_i2, gamma, beta, W_head, b_head):
    del W_root_i2, W_nbr_i2
    f32 = jnp.float32
    freqs = (1.0 / (10000.0 ** (jnp.arange(TDIM, dtype=f32) / TDIM)))
    ones_col = jnp.zeros((1, CP), f32).at[0, C].set(1.0)

    hu, hi = _tc1(
        x_user, x_item,
        time_user.reshape(N, 1), time_item.reshape(N, 1),
        batch_user.reshape(N, 1), batch_item.reshape(N, 1),
        seed_time.astype(f32).reshape(1, B), freqs.reshape(1, TDIM),
        _pad_out(W_enc_user), _pad_out(b_enc_user.reshape(1, C)) + ones_col,
        _pad_out(W_enc_item), _pad_out(b_enc_item.reshape(1, C)) + ones_col,
        _pad_out(W_time), _pad_out(b_time.reshape(1, C)),
    )

    def _redirect(dst, h):
        dl = dst.reshape(NC, NS, EPS)
        dummy = h + jnp.arange(NS, dtype=jnp.int32)[None, :, None]
        return jnp.where(dl < h, dl, dummy)

    su2i = edge_index_u2i[0].reshape(NC, NS, EPS)
    si2u = edge_index_i2u[0].reshape(NC, NS, EPS)
    du2i = edge_index_u2i[1].reshape(NC, NS, EPS)
    di2u = _redirect(edge_index_i2u[1], B)

    acc_i, acc_u = _sc1(hu, hi, su2i, si2u, du2i, di2u)

    nhi = _tc2(hi, acc_i, _pad_rows(_pad_out(W_root_i1)),
               _pad_rows(_pad_out(W_nbr_i1)), ones_col)

    acc_u2 = _sc2(nhi, si2u, di2u)
    if isinstance(acc_u2, (tuple, list)):
        (acc_u2,) = acc_u2

    return _tc3(hu, acc_u, acc_u2,
                _pad_rows(W_root_u1), _pad_rows(W_nbr_u1),
                W_root_u2, _pad_rows(W_nbr_u2),
                gamma.reshape(1, C), beta.reshape(1, C),
                W_head, b_head.reshape(1, OUT))
